# baseline (speedup 1.0000x reference)
import jax
import jax.numpy as jnp
from jax import lax
from jax.experimental import pallas as pl
from jax.experimental.pallas import tpu as pltpu
from jax.experimental.pallas import tpu_sc as plsc

B = 16384
NN_DIM = 64
MF_DIM = 32

_NC = 2
_NS = 16
_NW = _NC * _NS
_BPW = B // _NW
_ROWCHUNK = 32
_HALF = _BPW // 2

_BLK = 2048
_NBLK = B // _BLK


def _gather_body(user_hbm, item_hbm, nn_u_hbm, nn_i_hbm, mf_u_hbm, mf_i_hbm,
                 wmf_hbm, out_nnu, out_nni, out_mfc,
                 wv, mfu_v, mfi_v, cv, idx_v, sem):
    wid = lax.axis_index("s") * _NC + lax.axis_index("c")
    base = wid * _BPW
    pltpu.sync_copy(user_hbm.at[pl.ds(base, _BPW)], idx_v.at[0])
    pltpu.sync_copy(item_hbm.at[pl.ds(base, _BPW)], idx_v.at[1])
    pltpu.sync_copy(wmf_hbm, wv)

    def phase_a(c, _):
        copies = []
        for g in range(_ROWCHUNK // 16):
            uvec = idx_v[0, pl.ds(c * _ROWCHUNK + g * 16, 16)]
            vvec = idx_v[1, pl.ds(c * _ROWCHUNK + g * 16, 16)]
            for k in range(16):
                i = c * _ROWCHUNK + g * 16 + k
                copies.append(pltpu.async_copy(
                    nn_u_hbm.at[pl.ds(uvec[k], 1), :], out_nnu.at[pl.ds(base + i, 1), :], sem))
                copies.append(pltpu.async_copy(
                    nn_i_hbm.at[pl.ds(vvec[k], 1), :], out_nni.at[pl.ds(base + i, 1), :], sem))
        for cp in copies:
            cp.wait()
        return _

    lax.fori_loop(0, _BPW // _ROWCHUNK, phase_a, None)

    def phase_b(h, _):
        def gather_mf(c, _):
            copies = []
            for g in range(_ROWCHUNK // 16):
                uvec = idx_v[0, pl.ds(h * _HALF + c * _ROWCHUNK + g * 16, 16)]
                vvec = idx_v[1, pl.ds(h * _HALF + c * _ROWCHUNK + g * 16, 16)]
                for k in range(16):
                    i = c * _ROWCHUNK + g * 16 + k
                    copies.append(pltpu.async_copy(
                        mf_u_hbm.at[pl.ds(uvec[k], 1), :], mfu_v.at[pl.ds(i, 1), :], sem))
                    copies.append(pltpu.async_copy(
                        mf_i_hbm.at[pl.ds(vvec[k], 1), :], mfi_v.at[pl.ds(i, 1), :], sem))
            for cp in copies:
                cp.wait()
            return _

        lax.fori_loop(0, _HALF // _ROWCHUNK, gather_mf, None)

        wlo = wv[pl.ds(0, 16)]
        whi = wv[pl.ds(16, 16)]

        def reduce_g(g, _):
            rows = g * 16 + lax.iota(jnp.int32, 16)
            acc = jnp.zeros((16,), jnp.float32)
            for d in range(MF_DIM):
                w = wlo[d] if d < 16 else whi[d - 16]
                cols = jnp.full((16,), d, jnp.int32)
                vu = plsc.load_gather(mfu_v, [rows, cols])
                vi = plsc.load_gather(mfi_v, [rows, cols])
                acc = acc + vu * vi * w
            cv[pl.ds(h * _HALF + g * 16, 16)] = acc
            return _

        lax.fori_loop(0, _HALF // 16, reduce_g, None)
        return _

    lax.fori_loop(0, 2, phase_b, None)
    pltpu.sync_copy(cv, out_mfc.at[pl.ds(base, _BPW)])


def _sc_gather(user, item, nn_usr, nn_item, mf_usr, mf_item, wmf):
    mesh = plsc.VectorSubcoreMesh(core_axis_name="c", subcore_axis_name="s")
    f32 = jnp.float32
    return pl.kernel(
        _gather_body,
        out_type=[
            jax.ShapeDtypeStruct((B, NN_DIM), f32),
            jax.ShapeDtypeStruct((B, NN_DIM), f32),
            jax.ShapeDtypeStruct((B,), f32),
        ],
        mesh=mesh,
        scratch_types=[
            pltpu.VMEM((MF_DIM,), f32),
            pltpu.VMEM((_HALF, MF_DIM), f32),
            pltpu.VMEM((_HALF, MF_DIM), f32),
            pltpu.VMEM((_BPW,), f32),
            pltpu.VMEM((2, _BPW), jnp.int32),
            pltpu.SemaphoreType.DMA,
        ],
        compiler_params=pltpu.CompilerParams(needs_layout_passes=False),
    )(user, item, nn_usr, nn_item, mf_usr, mf_item, wmf)


def _mlp_body(nnu, nni, mfc, w1u, w1i, b1, w2, b2, w3, b3, wx, bo, out):
    hp = lax.Precision.HIGHEST
    f32 = jnp.float32
    x = jnp.dot(nnu[...], w1u[...], precision=hp, preferred_element_type=f32)
    x = x + jnp.dot(nni[...], w1i[...], precision=hp, preferred_element_type=f32)
    x = jnp.maximum(x + b1[...], 0.0)
    x = jnp.maximum(jnp.dot(x, w2[...], precision=hp, preferred_element_type=f32) + b2[...], 0.0)
    x = jnp.maximum(jnp.dot(x, w3[...], precision=hp, preferred_element_type=f32) + b3[...], 0.0)
    acc = jnp.sum(x * wx[...], axis=1) + mfc[...] + bo[0, 0]
    out[...] = acc


def kernel(user, item, mf_usr, mf_item, nn_usr, nn_item,
           fc1_w, fc1_b, fc2_w, fc2_b, fc3_w, fc3_b, neumf_w, neumf_b):
    user = user.astype(jnp.int32)
    item = item.astype(jnp.int32)
    wmf = neumf_w[0, :MF_DIM]
    nn_u, nn_i, mfc = _sc_gather(user, item, nn_usr, nn_item, mf_usr, mf_item, wmf)

    w1 = fc1_w.T
    w1u, w1i = w1[:NN_DIM], w1[NN_DIM:]
    w2 = fc2_w.T
    w3 = fc3_w.T
    wx = neumf_w[:, MF_DIM:]

    full = lambda shape: pl.BlockSpec(shape, lambda i: (0, 0))
    return pl.pallas_call(
        _mlp_body,
        grid=(_NBLK,),
        in_specs=[
            pl.BlockSpec((_BLK, NN_DIM), lambda i: (i, 0)),
            pl.BlockSpec((_BLK, NN_DIM), lambda i: (i, 0)),
            pl.BlockSpec((_BLK,), lambda i: (i,)),
            full((NN_DIM, 128)), full((NN_DIM, 128)), full((1, 128)),
            full((128, 64)), full((1, 64)),
            full((64, 32)), full((1, 32)),
            full((1, 32)), full((1, 1)),
        ],
        out_specs=pl.BlockSpec((_BLK,), lambda i: (i,)),
        out_shape=jax.ShapeDtypeStruct((B,), jnp.float32),
    )(nn_u, nn_i, mfc, w1u, w1i, fc1_b[None], w2, fc2_b[None],
      w3, fc3_b[None], wx, neumf_b[None])


# speedup vs baseline: 2.2848x; 2.2848x over previous
import jax
import jax.numpy as jnp
from jax import lax
from jax.experimental import pallas as pl
from jax.experimental.pallas import tpu as pltpu
from jax.experimental.pallas import tpu_sc as plsc

B = 16384
NN_DIM = 64
MF_DIM = 32
NROWS = 100000
PK = 128

_NC = 2
_NS = 16
_NW = _NC * _NS
_BPW = B // _NW
_ICH = 128
_NICH = _BPW // _ICH

_BLKR = 5000
_BLK = 2048
_NBLK = B // _BLK


def _pack_body(nn, mf, out):
    blk = nn[...].shape[0]
    out[...] = jnp.concatenate(
        [nn[...], mf[...], jnp.zeros((blk, PK - NN_DIM - MF_DIM), jnp.float32)],
        axis=1)


def _pack(nn, mf):
    return pl.pallas_call(
        _pack_body,
        grid=(NROWS // _BLKR,),
        in_specs=[
            pl.BlockSpec((_BLKR, NN_DIM), lambda i: (i, 0)),
            pl.BlockSpec((_BLKR, MF_DIM), lambda i: (i, 0)),
        ],
        out_specs=pl.BlockSpec((_BLKR, PK), lambda i: (i, 0)),
        out_shape=jax.ShapeDtypeStruct((NROWS, PK), jnp.float32),
    )(nn, mf)


def _gather_body(idx_hbm, tab_hbm, out_g, idx_v, dst, sem):
    wid = lax.axis_index("s") * _NC + lax.axis_index("c")
    base = wid * _BPW
    pltpu.sync_copy(idx_hbm.at[pl.ds(base, _BPW)], idx_v)
    copies = []
    for c in range(_NICH):
        sl = pl.ds(c * _ICH, _ICH)
        copies.append(pltpu.async_copy(tab_hbm.at[idx_v.at[sl]], dst.at[sl, :], sem))
    for cp in copies:
        cp.wait()
    pltpu.sync_copy(dst, out_g.at[pl.ds(base, _BPW), :])


def _sc_gather(idx, tab):
    mesh = plsc.VectorSubcoreMesh(core_axis_name="c", subcore_axis_name="s")
    return pl.kernel(
        _gather_body,
        out_type=jax.ShapeDtypeStruct((B, PK), jnp.float32),
        mesh=mesh,
        scratch_types=[
            pltpu.VMEM((_BPW,), jnp.int32),
            pltpu.VMEM((_BPW, PK), jnp.float32),
            pltpu.SemaphoreType.DMA,
        ],
        compiler_params=pltpu.CompilerParams(needs_layout_passes=False),
    )(idx, tab)


def _mlp_body(gu, gi, a1, c1, b1, w2, b2, w3, b3, wmfe, wx, bo, out):
    hp = lax.Precision.HIGHEST
    f32 = jnp.float32
    x = jnp.dot(gu[...], a1[...], precision=hp, preferred_element_type=f32)
    x = x + jnp.dot(gi[...], c1[...], precision=hp, preferred_element_type=f32)
    x = jnp.maximum(x + b1[...], 0.0)
    x = jnp.maximum(jnp.dot(x, w2[...], precision=hp, preferred_element_type=f32) + b2[...], 0.0)
    x = jnp.maximum(jnp.dot(x, w3[...], precision=hp, preferred_element_type=f32) + b3[...], 0.0)
    mf = jnp.sum(gu[...] * gi[...] * wmfe[...], axis=1)
    acc = jnp.sum(x * wx[...], axis=1) + mf + bo[0, 0]
    out[...] = acc


def kernel(user, item, mf_usr, mf_item, nn_usr, nn_item,
           fc1_w, fc1_b, fc2_w, fc2_b, fc3_w, fc3_b, neumf_w, neumf_b):
    user = user.astype(jnp.int32)
    item = item.astype(jnp.int32)

    tab_u = _pack(nn_usr, mf_usr)
    gu = _sc_gather(user, tab_u)
    tab_i = _pack(nn_item, mf_item)
    gi = _sc_gather(item, tab_i)

    w1 = fc1_w.T
    zpad = jnp.zeros((PK - NN_DIM, 128), jnp.float32)
    a1 = jnp.concatenate([w1[:NN_DIM], zpad], axis=0)
    c1 = jnp.concatenate([w1[NN_DIM:], zpad], axis=0)
    w2 = fc2_w.T
    w3 = fc3_w.T
    wmfe = jnp.concatenate(
        [jnp.zeros((1, NN_DIM), jnp.float32), neumf_w[:, :MF_DIM],
         jnp.zeros((1, PK - NN_DIM - MF_DIM), jnp.float32)], axis=1)
    wx = neumf_w[:, MF_DIM:]

    full = lambda shape: pl.BlockSpec(shape, lambda i: (0, 0))
    return pl.pallas_call(
        _mlp_body,
        grid=(_NBLK,),
        in_specs=[
            pl.BlockSpec((_BLK, PK), lambda i: (i, 0)),
            pl.BlockSpec((_BLK, PK), lambda i: (i, 0)),
            full((PK, 128)), full((PK, 128)), full((1, 128)),
            full((128, 64)), full((1, 64)),
            full((64, 32)), full((1, 32)),
            full((1, PK)), full((1, 32)), full((1, 1)),
        ],
        out_specs=pl.BlockSpec((_BLK,), lambda i: (i,)),
        out_shape=jax.ShapeDtypeStruct((B,), jnp.float32),
    )(gu, gi, a1, c1, fc1_b[None], w2, fc2_b[None],
      w3, fc3_b[None], wmfe, wx, neumf_b[None])


# speedup vs baseline: 3.3874x; 1.4826x over previous
import jax
import jax.numpy as jnp
from jax import lax
from jax.experimental import pallas as pl
from jax.experimental.pallas import tpu as pltpu
from jax.experimental.pallas import tpu_sc as plsc

B = 16384
NN_DIM = 64
MF_DIM = 32
NROWS = 100000
PK = 128

_NC = 2
_NS = 16
_NW = _NC * _NS
_BPW = B // _NW
_ICH = 128
_NICH = _BPW // _ICH

_BLKR = 2048
_BLK = 2048
_NBLK = B // _BLK


def _pack_body(nn, mf, out):
    nnr = nn[...].T
    mfr = mf[...].T
    out[...] = jnp.concatenate(
        [nnr, mfr, jnp.zeros((nnr.shape[0], PK - NN_DIM - MF_DIM), jnp.float32)],
        axis=1)


def _pack(nnT, mfT):
    return pl.pallas_call(
        _pack_body,
        grid=(pl.cdiv(NROWS, _BLKR),),
        in_specs=[
            pl.BlockSpec((NN_DIM, _BLKR), lambda i: (0, i)),
            pl.BlockSpec((MF_DIM, _BLKR), lambda i: (0, i)),
        ],
        out_specs=pl.BlockSpec((_BLKR, PK), lambda i: (i, 0)),
        out_shape=jax.ShapeDtypeStruct((NROWS, PK), jnp.float32),
    )(nnT, mfT)


def _gather_body(idx_hbm, tab_hbm, out_g, idx_v, dst, sem):
    wid = lax.axis_index("s") * _NC + lax.axis_index("c")
    base = wid * _BPW
    pltpu.sync_copy(idx_hbm.at[pl.ds(base, _BPW)], idx_v)
    copies = []
    for c in range(_NICH):
        sl = pl.ds(c * _ICH, _ICH)
        copies.append(pltpu.async_copy(tab_hbm.at[idx_v.at[sl]], dst.at[sl, :], sem))
    for cp in copies:
        cp.wait()
    pltpu.sync_copy(dst, out_g.at[pl.ds(base, _BPW), :])


def _sc_gather(idx, tab):
    mesh = plsc.VectorSubcoreMesh(core_axis_name="c", subcore_axis_name="s")
    return pl.kernel(
        _gather_body,
        out_type=jax.ShapeDtypeStruct((B, PK), jnp.float32),
        mesh=mesh,
        scratch_types=[
            pltpu.VMEM((_BPW,), jnp.int32),
            pltpu.VMEM((_BPW, PK), jnp.float32),
            pltpu.SemaphoreType.DMA,
        ],
        compiler_params=pltpu.CompilerParams(needs_layout_passes=False),
    )(idx, tab)


def _mlp_body(gu, gi, a1, c1, b1, w2, b2, w3, b3, wmfe, wx, bo, out):
    hp = lax.Precision.HIGHEST
    f32 = jnp.float32
    x = jnp.dot(gu[...], a1[...], precision=hp, preferred_element_type=f32)
    x = x + jnp.dot(gi[...], c1[...], precision=hp, preferred_element_type=f32)
    x = jnp.maximum(x + b1[...], 0.0)
    x = jnp.maximum(jnp.dot(x, w2[...], precision=hp, preferred_element_type=f32) + b2[...], 0.0)
    x = jnp.maximum(jnp.dot(x, w3[...], precision=hp, preferred_element_type=f32) + b3[...], 0.0)
    mf = jnp.sum(gu[...] * gi[...] * wmfe[...], axis=1)
    acc = jnp.sum(x * wx[...], axis=1) + mf + bo[0, 0]
    out[...] = acc


def kernel(user, item, mf_usr, mf_item, nn_usr, nn_item,
           fc1_w, fc1_b, fc2_w, fc2_b, fc3_w, fc3_b, neumf_w, neumf_b):
    user = user.astype(jnp.int32)
    item = item.astype(jnp.int32)

    tab_u = _pack(nn_usr.T, mf_usr.T)
    gu = _sc_gather(user, tab_u)
    tab_i = _pack(nn_item.T, mf_item.T)
    gi = _sc_gather(item, tab_i)

    w1 = fc1_w.T
    zpad = jnp.zeros((PK - NN_DIM, 128), jnp.float32)
    a1 = jnp.concatenate([w1[:NN_DIM], zpad], axis=0)
    c1 = jnp.concatenate([w1[NN_DIM:], zpad], axis=0)
    w2 = fc2_w.T
    w3 = fc3_w.T
    wmfe = jnp.concatenate(
        [jnp.zeros((1, NN_DIM), jnp.float32), neumf_w[:, :MF_DIM],
         jnp.zeros((1, PK - NN_DIM - MF_DIM), jnp.float32)], axis=1)
    wx = neumf_w[:, MF_DIM:]

    full = lambda shape: pl.BlockSpec(shape, lambda i: (0, 0))
    return pl.pallas_call(
        _mlp_body,
        grid=(_NBLK,),
        in_specs=[
            pl.BlockSpec((_BLK, PK), lambda i: (i, 0)),
            pl.BlockSpec((_BLK, PK), lambda i: (i, 0)),
            full((PK, 128)), full((PK, 128)), full((1, 128)),
            full((128, 64)), full((1, 64)),
            full((64, 32)), full((1, 32)),
            full((1, PK)), full((1, 32)), full((1, 1)),
        ],
        out_specs=pl.BlockSpec((_BLK,), lambda i: (i,)),
        out_shape=jax.ShapeDtypeStruct((B,), jnp.float32),
    )(gu, gi, a1, c1, fc1_b[None], w2, fc2_b[None],
      w3, fc3_b[None], wmfe, wx, neumf_b[None])


# speedup vs baseline: 4.9535x; 1.4623x over previous
import jax
import jax.numpy as jnp
from jax import lax
from jax.experimental import pallas as pl
from jax.experimental.pallas import tpu as pltpu
from jax.experimental.pallas import tpu_sc as plsc

B = 16384
NN_DIM = 64
MF_DIM = 32
NROWS = 100000
PK = 128

_NC = 2
_NS = 16
_NW = _NC * _NS
_BPW = B // _NW
_ICH = 128
_NICH = _BPW // _ICH

_BLKR = 8192
_BLK = 8192
_NBLK = B // _BLK


def _pack_body(nn, mf, out):
    nnr = nn[...].T
    mfr = mf[...].T
    out[...] = jnp.concatenate(
        [nnr, mfr, jnp.zeros((nnr.shape[0], PK - NN_DIM - MF_DIM), jnp.float32)],
        axis=1)


def _pack(nnT, mfT):
    return pl.pallas_call(
        _pack_body,
        grid=(pl.cdiv(NROWS, _BLKR),),
        in_specs=[
            pl.BlockSpec((NN_DIM, _BLKR), lambda i: (0, i)),
            pl.BlockSpec((MF_DIM, _BLKR), lambda i: (0, i)),
        ],
        out_specs=pl.BlockSpec((_BLKR, PK), lambda i: (i, 0)),
        out_shape=jax.ShapeDtypeStruct((NROWS, PK), jnp.float32),
    )(nnT, mfT)


def _gather_body(idx_hbm, tab_hbm, out_g, idx_v, dst, sem):
    wid = lax.axis_index("s") * _NC + lax.axis_index("c")
    base = wid * _BPW
    pltpu.sync_copy(idx_hbm.at[pl.ds(base, _BPW)], idx_v)
    copies = []
    for c in range(_NICH):
        sl = pl.ds(c * _ICH, _ICH)
        copies.append(pltpu.async_copy(tab_hbm.at[idx_v.at[sl]], dst.at[sl, :], sem))
    for cp in copies:
        cp.wait()
    pltpu.sync_copy(dst, out_g.at[pl.ds(base, _BPW), :])


def _sc_gather(idx, tab):
    mesh = plsc.VectorSubcoreMesh(core_axis_name="c", subcore_axis_name="s")
    return pl.kernel(
        _gather_body,
        out_type=jax.ShapeDtypeStruct((B, PK), jnp.float32),
        mesh=mesh,
        scratch_types=[
            pltpu.VMEM((_BPW,), jnp.int32),
            pltpu.VMEM((_BPW, PK), jnp.float32),
            pltpu.SemaphoreType.DMA,
        ],
        compiler_params=pltpu.CompilerParams(needs_layout_passes=False),
    )(idx, tab)


def _mlp_body(gu, gi, a1, c1, b1, w2, b2, w3, b3, wmfe, wx, bo, out):
    hp = lax.Precision.DEFAULT
    f32 = jnp.float32
    x = jnp.dot(gu[...], a1[...], precision=hp, preferred_element_type=f32)
    x = x + jnp.dot(gi[...], c1[...], precision=hp, preferred_element_type=f32)
    x = jnp.maximum(x + b1[...], 0.0)
    x = jnp.maximum(jnp.dot(x, w2[...], precision=hp, preferred_element_type=f32) + b2[...], 0.0)
    x = jnp.maximum(jnp.dot(x, w3[...], precision=hp, preferred_element_type=f32) + b3[...], 0.0)
    mf = jnp.sum(gu[...] * gi[...] * wmfe[...], axis=1)
    acc = jnp.sum(x * wx[...], axis=1) + mf + bo[0, 0]
    out[...] = acc


def kernel(user, item, mf_usr, mf_item, nn_usr, nn_item,
           fc1_w, fc1_b, fc2_w, fc2_b, fc3_w, fc3_b, neumf_w, neumf_b):
    user = user.astype(jnp.int32)
    item = item.astype(jnp.int32)

    tab_u = _pack(nn_usr.T, mf_usr.T)
    gu = _sc_gather(user, tab_u)
    tab_i = _pack(nn_item.T, mf_item.T)
    gi = _sc_gather(item, tab_i)

    w1 = fc1_w.T
    zpad = jnp.zeros((PK - NN_DIM, 128), jnp.float32)
    a1 = jnp.concatenate([w1[:NN_DIM], zpad], axis=0)
    c1 = jnp.concatenate([w1[NN_DIM:], zpad], axis=0)
    w2 = fc2_w.T
    w3 = fc3_w.T
    wmfe = jnp.concatenate(
        [jnp.zeros((1, NN_DIM), jnp.float32), neumf_w[:, :MF_DIM],
         jnp.zeros((1, PK - NN_DIM - MF_DIM), jnp.float32)], axis=1)
    wx = neumf_w[:, MF_DIM:]

    full = lambda shape: pl.BlockSpec(shape, lambda i: (0, 0))
    return pl.pallas_call(
        _mlp_body,
        grid=(_NBLK,),
        in_specs=[
            pl.BlockSpec((_BLK, PK), lambda i: (i, 0)),
            pl.BlockSpec((_BLK, PK), lambda i: (i, 0)),
            full((PK, 128)), full((PK, 128)), full((1, 128)),
            full((128, 64)), full((1, 64)),
            full((64, 32)), full((1, 32)),
            full((1, PK)), full((1, 32)), full((1, 1)),
        ],
        out_specs=pl.BlockSpec((_BLK,), lambda i: (i,)),
        out_shape=jax.ShapeDtypeStruct((B,), jnp.float32),
    )(gu, gi, a1, c1, fc1_b[None], w2, fc2_b[None],
      w3, fc3_b[None], wmfe, wx, neumf_b[None])
